# baseline (speedup 1.0000x reference)
import jax
import jax.numpy as jnp
from jax import lax
from jax.experimental import pallas as pl
from jax.experimental.pallas import tpu as pltpu
from jax.experimental.pallas import tpu_sc as plsc

N_CHARS = 1000
EMB_DIM = 32
BATCH = 16384
SEQ = 200

_INFO = plsc.get_sparse_core_info()
_NC = _INFO.num_cores
_NS = _INFO.num_subcores
_NW = _NC * _NS

_TOTAL = BATCH * SEQ
_PER_W = _TOTAL // _NW
_CHUNK = 1600
_STEPS = _PER_W // _CHUNK


def _gather_kernel(idx_hbm, table_hbm, out_hbm,
                   idx0, idx1, rows0, rows1,
                   sem_i0, sem_i1, sem_g0, sem_g1, sem_o0, sem_o1):
    wid = lax.axis_index("s") * _NC + lax.axis_index("c")
    base = wid * _PER_W

    def idx_load(g, buf, sem):
        pltpu.async_copy(idx_hbm.at[pl.ds(base + g * _CHUNK, _CHUNK)], buf, sem)

    def idx_wait(buf, sem):
        pltpu.make_async_copy(
            idx_hbm.at[pl.ds(base, _CHUNK)], buf, sem).wait()

    def gather(buf_idx, buf_rows, sem):
        pltpu.async_copy(table_hbm.at[buf_idx], buf_rows, sem)

    def gather_wait(buf_idx, buf_rows, sem):
        pltpu.make_async_copy(table_hbm.at[buf_idx], buf_rows, sem).wait()

    def out_write(g, buf, sem):
        pltpu.async_copy(buf, out_hbm.at[pl.ds(base + g * _CHUNK, _CHUNK)], sem)

    def out_wait(buf, sem):
        pltpu.make_async_copy(
            buf, out_hbm.at[pl.ds(base, _CHUNK)], sem).wait()

    idx_load(0, idx0, sem_i0)
    idx_load(1, idx1, sem_i1)
    out_write(1, rows1, sem_o1)
    idx_wait(idx0, sem_i0)
    gather(idx0, rows0, sem_g0)

    def body(p, carry):
        g0 = 2 * p
        gather_wait(idx0, rows0, sem_g0)
        idx_load(g0 + 2, idx0, sem_i0)
        out_wait(rows1, sem_o1)
        idx_wait(idx1, sem_i1)
        gather(idx1, rows1, sem_g1)
        out_write(g0, rows0, sem_o0)
        gather_wait(idx1, rows1, sem_g1)
        idx_load(g0 + 3, idx1, sem_i1)
        out_wait(rows0, sem_o0)
        idx_wait(idx0, sem_i0)
        gather(idx0, rows0, sem_g0)
        out_write(g0 + 1, rows1, sem_o1)
        return carry

    lax.fori_loop(0, (_STEPS - 2) // 2, body, 0, unroll=False)

    g_last = _STEPS - 2
    gather_wait(idx0, rows0, sem_g0)
    out_wait(rows1, sem_o1)
    idx_wait(idx1, sem_i1)
    gather(idx1, rows1, sem_g1)
    out_write(g_last, rows0, sem_o0)
    gather_wait(idx1, rows1, sem_g1)
    out_wait(rows0, sem_o0)
    out_write(g_last + 1, rows1, sem_o1)
    out_wait(rows1, sem_o1)


@jax.jit
def kernel(sentence, table):
    idx = sentence.reshape(_TOTAL)
    mesh = plsc.VectorSubcoreMesh(core_axis_name="c", subcore_axis_name="s")
    flat = pl.kernel(
        _gather_kernel,
        out_type=jax.ShapeDtypeStruct((_TOTAL, EMB_DIM), jnp.float32),
        mesh=mesh,
        scratch_types=[
            pltpu.VMEM((_CHUNK,), jnp.int32),
            pltpu.VMEM((_CHUNK,), jnp.int32),
            pltpu.VMEM((_CHUNK, EMB_DIM), jnp.float32),
            pltpu.VMEM((_CHUNK, EMB_DIM), jnp.float32),
            pltpu.SemaphoreType.DMA,
            pltpu.SemaphoreType.DMA,
            pltpu.SemaphoreType.DMA,
            pltpu.SemaphoreType.DMA,
            pltpu.SemaphoreType.DMA,
            pltpu.SemaphoreType.DMA,
        ],
        compiler_params=pltpu.CompilerParams(use_tc_tiling_on_sc=False),
    )(idx, table)
    return flat.reshape(BATCH, SEQ, EMB_DIM)
